# baseline (speedup 1.0000x reference)
import jax
import jax.numpy as jnp
from jax import lax
from jax.experimental import pallas as pl
from jax.experimental.pallas import tpu as pltpu
from jax.experimental.pallas import tpu_sc as plsc

T = 8192
KC = 8
DC = 256
CW = KC + 2 * DC * KC
BT = 256
RC = 64
NC, NS, L = 2, 16, 16
NW = NC * NS
TPW = T // NW

_SC_PARAMS = pltpu.CompilerParams(use_tc_tiling_on_sc=False,
                                  needs_layout_passes=False)


def _sel_body(lg_hbm, sel_hbm, lg_v, out_v, sem):
    c = lax.axis_index("c")
    s = lax.axis_index("s")
    wid = s * NC + c
    base = wid * (TPW * KC)
    pltpu.sync_copy(lg_hbm.at[pl.ds(base, TPW * KC)], lg_v)

    def chunk(u, carry):
        tj = u // 8
        uu = u % 8
        off = tj * 1024 + uu * L
        lk = [lg_v[pl.ds(off + k * 128, L)] for k in range(KC)]
        m = lk[0]
        for k in range(1, KC):
            m = jnp.maximum(m, lk[k])
        ek = [jnp.exp(x - m) for x in lk]
        ssum = ek[0]
        for k in range(1, KC):
            ssum = ssum + ek[k]
        r = 1.0 / ssum
        for k in range(KC):
            out_v[pl.ds(off + k * 128, L)] = ek[k] * r
        return carry
    lax.fori_loop(0, 16, chunk, 0)
    pltpu.sync_copy(out_v, sel_hbm.at[pl.ds(base, TPW * KC)])


def _sel_head(lg_flat):
    mesh = plsc.VectorSubcoreMesh(core_axis_name="c", subcore_axis_name="s",
                                  num_cores=NC, num_subcores=NS)
    return pl.kernel(
        _sel_body,
        out_type=jax.ShapeDtypeStruct((T * KC,), jnp.float32),
        mesh=mesh,
        compiler_params=_SC_PARAMS,
        scratch_types=[
            pltpu.VMEM((TPW * KC,), jnp.float32),
            pltpu.VMEM((TPW * KC,), jnp.float32),
            pltpu.SemaphoreType.DMA,
        ],
    )(lg_flat)


def _route_body(xt_ref, compT_ref):
    lg = [xt_ref[pl.ds(k, 1), :] for k in range(KC)]
    am = jnp.zeros((1, BT), jnp.int32)
    bm = lg[0]
    for k in range(1, KC):
        gt = lg[k] > bm
        am = jnp.where(gt, k, am)
        bm = jnp.maximum(bm, lg[k])
    for rr in range(0, 2 * DC, RC):
        acc = jnp.where(am == 0, xt_ref[pl.ds(KC + rr, RC), :], 0.0)
        for a in range(1, KC):
            acc = jnp.where(am == a,
                            xt_ref[pl.ds(KC + 2 * DC * a + rr, RC), :], acc)
        compT_ref[pl.ds(rr, RC), :] = acc


def _route(xt):
    return pl.pallas_call(
        _route_body,
        grid=(T // BT,),
        in_specs=[pl.BlockSpec((CW, BT), lambda i: (0, i))],
        out_specs=pl.BlockSpec((2 * DC, BT), lambda i: (0, i)),
        out_shape=jax.ShapeDtypeStruct((2 * DC, T), jnp.float32),
    )(xt)


def _sample_body(compT_ref, eps_ref, comp_ref, out_ref):
    cb = jnp.transpose(compT_ref[...])
    comp_ref[...] = cb
    mean = cb[:, :DC]
    logvar = cb[:, DC:]
    out_ref[...] = mean + jnp.exp(0.5 * logvar) * eps_ref[...]


def _sample(compT, eps):
    return pl.pallas_call(
        _sample_body,
        grid=(T // BT,),
        in_specs=[pl.BlockSpec((2 * DC, BT), lambda i: (0, i)),
                  pl.BlockSpec((BT, DC), lambda i: (i, 0))],
        out_specs=[pl.BlockSpec((BT, 2 * DC), lambda i: (i, 0)),
                   pl.BlockSpec((BT, DC), lambda i: (i, 0))],
        out_shape=[jax.ShapeDtypeStruct((T, 2 * DC), jnp.float32),
                   jax.ShapeDtypeStruct((T, DC), jnp.float32)],
    )(compT, eps)


def kernel(raw_params):
    xt = raw_params.T
    lg_flat = (xt[:KC, :].reshape(KC, T // 128, 128)
               .transpose(1, 0, 2).reshape(T * KC))
    sel_flat = _sel_head(lg_flat)
    selector_params = (sel_flat.reshape(T // 128, KC, 128)
                       .transpose(1, 0, 2).reshape(KC, T)).T
    compT = _route(xt)
    eps = jax.random.normal(jax.random.key(42), (T, DC), dtype=jnp.float32)
    component_params, samples = _sample(compT, eps)
    return (selector_params, component_params, samples)
